# Initial kernel scaffold; baseline (speedup 1.0000x reference)
import jax
import jax.numpy as jnp
from jax.experimental import pallas as pl


def kernel(x, edge_index, W1l, W1r, b1, W2l, W2r, b2):
    raise NotImplementedError("write your pallas kernel here")


# speedup vs baseline: 14.3749x; 14.3749x over previous
import functools

import jax
import jax.numpy as jnp
from jax import lax
from jax.experimental import pallas as pl
from jax.experimental.pallas import tpu as pltpu
from jax.experimental.pallas import tpu_sc as plsc

N = 10000
D = 128
HID = 8
NCLS = 16
TBW = 16
N_PAD = 10240
E = 320000
NC = 2
NS = 16
NW = NC * NS
CHUNK = 128
CPT = 80
E_PAD = NW * CPT * CHUNK
ROWS_PT = N_PAD // NS

_mesh = plsc.VectorSubcoreMesh(core_axis_name="c", subcore_axis_name="s")


@functools.partial(
    pl.kernel,
    out_type=jax.ShapeDtypeStruct((NC, N_PAD, TBW), jnp.float32),
    mesh=_mesh,
    compiler_params=pltpu.CompilerParams(use_tc_tiling_on_sc=False),
    scratch_types=[
        pltpu.VMEM((CPT, CHUNK), jnp.int32),
        pltpu.VMEM((CPT, CHUNK), jnp.int32),
        pltpu.VMEM((CHUNK, TBW), jnp.float32),
        pltpu.VMEM((CHUNK, TBW), jnp.float32),
        pltpu.VMEM_SHARED((N_PAD, TBW), jnp.float32),
        pltpu.SemaphoreType.DMA,
        pltpu.SemaphoreType.DMA,
    ],
)
def _edge_agg(table_hbm, src_hbm, dst_hbm, zeros_hbm, out_hbm,
              src_v, dst_v, rows0, rows1, acc, sem0, sem1):
    c = lax.axis_index("c")
    s = lax.axis_index("s")
    wid = s * NC + c
    row0 = s * ROWS_PT
    pltpu.sync_copy(zeros_hbm.at[pl.ds(row0, ROWS_PT)],
                    acc.at[pl.ds(row0, ROWS_PT)])
    pltpu.sync_copy(src_hbm.at[wid], src_v)
    pltpu.sync_copy(dst_hbm.at[wid], dst_v)
    plsc.subcore_barrier()

    @pl.loop(0, CPT, step=2)
    def _(j):
        g0 = pltpu.async_copy(table_hbm.at[src_v.at[j]], rows0, sem0)
        g1 = pltpu.async_copy(table_hbm.at[src_v.at[j + 1]], rows1, sem1)
        g0.wait()
        pltpu.sync_copy(rows0, acc.at[dst_v.at[j]], add=True)
        g1.wait()
        pltpu.sync_copy(rows1, acc.at[dst_v.at[j + 1]], add=True)

    plsc.subcore_barrier()
    pltpu.sync_copy(acc.at[pl.ds(row0, ROWS_PT)],
                    out_hbm.at[c, pl.ds(row0, ROWS_PT)])


BLK = 1024
GRID = N_PAD // BLK


def _pre_body(x_ref, wl_ref, wr_ref, b1_ref, t1_ref, r1_ref):
    xb = x_ref[...]
    t = jnp.dot(xb, wl_ref[...], preferred_element_type=jnp.float32)
    col = lax.broadcasted_iota(jnp.int32, (BLK, TBW), 1)
    t1_ref[...] = jnp.where(col == HID, t + 1.0, t)
    r1_ref[...] = (jnp.dot(xb, wr_ref[...], preferred_element_type=jnp.float32)
                   + b1_ref[...])


_pre = pl.pallas_call(
    _pre_body,
    grid=(GRID,),
    in_specs=[pl.BlockSpec((BLK, D), lambda i: (i, 0)),
              pl.BlockSpec((D, TBW), lambda i: (0, 0)),
              pl.BlockSpec((D, HID), lambda i: (0, 0)),
              pl.BlockSpec((1, HID), lambda i: (0, 0))],
    out_specs=[pl.BlockSpec((BLK, TBW), lambda i: (i, 0)),
               pl.BlockSpec((BLK, HID), lambda i: (i, 0))],
    out_shape=[jax.ShapeDtypeStruct((N_PAD, TBW), jnp.float32),
               jax.ShapeDtypeStruct((N_PAD, HID), jnp.float32)],
)


def _mid_body(p_ref, r1_ref, wl_ref, wr_ref, b2_ref, t2_ref, r2_ref, den_ref):
    ssum = p_ref[0] + p_ref[1]
    den = jnp.maximum(ssum[:, HID:HID + 1], 1.0)
    h = ssum[:, :HID] / den + r1_ref[...]
    h = jnp.where(h > 0, h, jnp.exp(jnp.minimum(h, 0.0)) - 1.0)
    t2_ref[...] = jnp.dot(h, wl_ref[...], preferred_element_type=jnp.float32)
    r2_ref[...] = (jnp.dot(h, wr_ref[...], preferred_element_type=jnp.float32)
                   + b2_ref[...])
    den_ref[...] = den


_mid = pl.pallas_call(
    _mid_body,
    grid=(GRID,),
    in_specs=[pl.BlockSpec((NC, BLK, TBW), lambda i: (0, i, 0)),
              pl.BlockSpec((BLK, HID), lambda i: (i, 0)),
              pl.BlockSpec((HID, NCLS), lambda i: (0, 0)),
              pl.BlockSpec((HID, NCLS), lambda i: (0, 0)),
              pl.BlockSpec((1, NCLS), lambda i: (0, 0))],
    out_specs=[pl.BlockSpec((BLK, NCLS), lambda i: (i, 0)),
               pl.BlockSpec((BLK, NCLS), lambda i: (i, 0)),
               pl.BlockSpec((BLK, 1), lambda i: (i, 0))],
    out_shape=[jax.ShapeDtypeStruct((N_PAD, NCLS), jnp.float32),
               jax.ShapeDtypeStruct((N_PAD, NCLS), jnp.float32),
               jax.ShapeDtypeStruct((N_PAD, 1), jnp.float32)],
)


def _fin_body(p_ref, den_ref, r2_ref, o_ref):
    z = (p_ref[0] + p_ref[1]) / den_ref[...] + r2_ref[...]
    m = jnp.max(z, axis=1, keepdims=True)
    e = jnp.exp(z - m)
    lse = jnp.log(jnp.sum(e, axis=1, keepdims=True))
    o_ref[...] = z - m - lse


_fin = pl.pallas_call(
    _fin_body,
    grid=(GRID,),
    in_specs=[pl.BlockSpec((NC, BLK, NCLS), lambda i: (0, i, 0)),
              pl.BlockSpec((BLK, 1), lambda i: (i, 0)),
              pl.BlockSpec((BLK, NCLS), lambda i: (i, 0))],
    out_specs=pl.BlockSpec((BLK, NCLS), lambda i: (i, 0)),
    out_shape=jax.ShapeDtypeStruct((N_PAD, NCLS), jnp.float32),
)


def kernel(x, edge_index, W1l, W1r, b1, W2l, W2r, b2):
    src = edge_index[0].astype(jnp.int32)
    dst = edge_index[1].astype(jnp.int32)
    pad_e = E_PAD - E
    src3 = jnp.concatenate(
        [src, jnp.zeros((pad_e,), jnp.int32)]).reshape(NW, CPT, CHUNK)
    dst3 = jnp.concatenate(
        [dst, jnp.full((pad_e,), N, jnp.int32)]).reshape(NW, CPT, CHUNK)
    x_pad = jnp.pad(x, ((0, N_PAD - N), (0, 0)))
    wl1 = jnp.pad(W1l, ((0, 0), (0, TBW - HID)))
    zeros_tab = jnp.zeros((N_PAD, TBW), jnp.float32)

    T1, R1 = _pre(x_pad, wl1, W1r, b1.reshape(1, HID))
    P1 = _edge_agg(T1, src3, dst3, zeros_tab)
    T2, R2, den = _mid(P1, R1, W2l, W2r, b2.reshape(1, NCLS))
    P2 = _edge_agg(T2, src3, dst3, zeros_tab)
    out = _fin(P2, den, R2)
    return out[:N]


# speedup vs baseline: 15.6286x; 1.0872x over previous
import functools

import jax
import jax.numpy as jnp
from jax import lax
from jax.experimental import pallas as pl
from jax.experimental.pallas import tpu as pltpu
from jax.experimental.pallas import tpu_sc as plsc

N = 10000
D = 128
HID = 8
NCLS = 16
TBW = 16
N_PAD = 10240
E = 320000
NC = 2
NS = 16
NW = NC * NS
CHUNK = 128
CPT = 80
E_PAD = NW * CPT * CHUNK
ROWS_PT = N_PAD // NS

_mesh = plsc.VectorSubcoreMesh(core_axis_name="c", subcore_axis_name="s")


@functools.partial(
    pl.kernel,
    out_type=jax.ShapeDtypeStruct((NC, N_PAD, TBW), jnp.float32),
    mesh=_mesh,
    compiler_params=pltpu.CompilerParams(use_tc_tiling_on_sc=False),
    scratch_types=[
        pltpu.VMEM((CPT, CHUNK), jnp.int32),
        pltpu.VMEM((CPT, CHUNK), jnp.int32),
        pltpu.VMEM((CHUNK, TBW), jnp.float32),
        pltpu.VMEM((CHUNK, TBW), jnp.float32),
        pltpu.VMEM_SHARED((N_PAD, TBW), jnp.float32),
        pltpu.SemaphoreType.DMA,
        pltpu.SemaphoreType.DMA,
    ],
)
def _edge_agg(table_hbm, edges_hbm, zeros_hbm, out_hbm,
              src_v, dst_v, rows0, rows1, acc, sem0, sem1):
    c = lax.axis_index("c")
    s = lax.axis_index("s")
    wid = s * NC + c
    row0 = s * ROWS_PT
    pltpu.sync_copy(zeros_hbm.at[pl.ds(row0, ROWS_PT)],
                    acc.at[pl.ds(row0, ROWS_PT)])
    pltpu.sync_copy(edges_hbm.at[0, wid], src_v)
    pltpu.sync_copy(edges_hbm.at[1, wid], dst_v)
    plsc.subcore_barrier()

    @pl.loop(0, CPT, step=2)
    def _(j):
        g0 = pltpu.async_copy(table_hbm.at[src_v.at[j]], rows0, sem0)
        g1 = pltpu.async_copy(table_hbm.at[src_v.at[j + 1]], rows1, sem1)
        g0.wait()
        pltpu.sync_copy(rows0, acc.at[dst_v.at[j]], add=True)
        g1.wait()
        pltpu.sync_copy(rows1, acc.at[dst_v.at[j + 1]], add=True)

    plsc.subcore_barrier()
    pltpu.sync_copy(acc.at[pl.ds(row0, ROWS_PT)],
                    out_hbm.at[c, pl.ds(row0, ROWS_PT)])


BLK = 2000
GRID = N // BLK


def _pre_body(x_ref, wl_ref, wr_ref, b1_ref, t1_ref, r1_ref):
    xb = x_ref[...]
    t = jnp.dot(xb, wl_ref[...], preferred_element_type=jnp.float32)
    col = lax.broadcasted_iota(jnp.int32, (BLK, TBW), 1)
    t1_ref[...] = jnp.where(col == HID, t + 1.0, t)
    r1_ref[...] = (jnp.dot(xb, wr_ref[...], preferred_element_type=jnp.float32)
                   + b1_ref[...])


_pre = pl.pallas_call(
    _pre_body,
    grid=(GRID,),
    in_specs=[pl.BlockSpec((BLK, D), lambda i: (i, 0)),
              pl.BlockSpec((D, TBW), lambda i: (0, 0)),
              pl.BlockSpec((D, HID), lambda i: (0, 0)),
              pl.BlockSpec((1, HID), lambda i: (0, 0))],
    out_specs=[pl.BlockSpec((BLK, TBW), lambda i: (i, 0)),
               pl.BlockSpec((BLK, HID), lambda i: (i, 0))],
    out_shape=[jax.ShapeDtypeStruct((N_PAD, TBW), jnp.float32),
               jax.ShapeDtypeStruct((N_PAD, HID), jnp.float32)],
)


def _mid_body(p_ref, r1_ref, wl_ref, wr_ref, b2_ref, t2_ref, r2_ref, den_ref):
    ssum = p_ref[0] + p_ref[1]
    den = jnp.maximum(ssum[:, HID:HID + 1], 1.0)
    h = ssum[:, :HID] / den + r1_ref[...]
    h = jnp.where(h > 0, h, jnp.exp(jnp.minimum(h, 0.0)) - 1.0)
    t2_ref[...] = jnp.dot(h, wl_ref[...], preferred_element_type=jnp.float32)
    r2_ref[...] = (jnp.dot(h, wr_ref[...], preferred_element_type=jnp.float32)
                   + b2_ref[...])
    den_ref[...] = den


_mid = pl.pallas_call(
    _mid_body,
    grid=(GRID,),
    in_specs=[pl.BlockSpec((NC, BLK, TBW), lambda i: (0, i, 0)),
              pl.BlockSpec((BLK, HID), lambda i: (i, 0)),
              pl.BlockSpec((HID, NCLS), lambda i: (0, 0)),
              pl.BlockSpec((HID, NCLS), lambda i: (0, 0)),
              pl.BlockSpec((1, NCLS), lambda i: (0, 0))],
    out_specs=[pl.BlockSpec((BLK, NCLS), lambda i: (i, 0)),
               pl.BlockSpec((BLK, NCLS), lambda i: (i, 0)),
               pl.BlockSpec((BLK, 1), lambda i: (i, 0))],
    out_shape=[jax.ShapeDtypeStruct((N_PAD, NCLS), jnp.float32),
               jax.ShapeDtypeStruct((N_PAD, NCLS), jnp.float32),
               jax.ShapeDtypeStruct((N_PAD, 1), jnp.float32)],
)


def _fin_body(p_ref, den_ref, r2_ref, o_ref):
    z = (p_ref[0] + p_ref[1]) / den_ref[...] + r2_ref[...]
    m = jnp.max(z, axis=1, keepdims=True)
    e = jnp.exp(z - m)
    lse = jnp.log(jnp.sum(e, axis=1, keepdims=True))
    o_ref[...] = z - m - lse


_fin = pl.pallas_call(
    _fin_body,
    grid=(GRID,),
    in_specs=[pl.BlockSpec((NC, BLK, NCLS), lambda i: (0, i, 0)),
              pl.BlockSpec((BLK, 1), lambda i: (i, 0)),
              pl.BlockSpec((BLK, NCLS), lambda i: (i, 0))],
    out_specs=pl.BlockSpec((BLK, NCLS), lambda i: (i, 0)),
    out_shape=jax.ShapeDtypeStruct((N, NCLS), jnp.float32),
)


def kernel(x, edge_index, W1l, W1r, b1, W2l, W2r, b2):
    ei = edge_index.astype(jnp.int32)
    ei4 = jnp.pad(ei, ((0, 0), (0, E_PAD - E)),
                  constant_values=N).reshape(2, NW, CPT, CHUNK)
    wl1 = jnp.pad(W1l, ((0, 0), (0, TBW - HID)))
    zeros_tab = jnp.zeros((N_PAD, TBW), jnp.float32)

    T1, R1 = _pre(x, wl1, W1r, b1.reshape(1, HID))
    P1 = _edge_agg(T1, ei4, zeros_tab)
    T2, R2, den = _mid(P1, R1, W2l, W2r, b2.reshape(1, NCLS))
    P2 = _edge_agg(T2, ei4, zeros_tab)
    return _fin(P2, den, R2)


# speedup vs baseline: 16.8334x; 1.0771x over previous
import functools

import jax
import jax.numpy as jnp
from jax import lax
from jax.experimental import pallas as pl
from jax.experimental.pallas import tpu as pltpu
from jax.experimental.pallas import tpu_sc as plsc

N = 10000
D = 128
HID = 8
NCLS = 16
TBW = 16
N_PAD = 10240
E = 320000
NC = 2
NS = 16
NW = NC * NS
CHUNK = 128
CPT0 = 96
CPT1 = 64
NCHUNKS = NS * (CPT0 + CPT1)
E_PAD = NCHUNKS * CHUNK
ROWS_PT = N_PAD // NS

_mesh = plsc.VectorSubcoreMesh(core_axis_name="c", subcore_axis_name="s")


@functools.partial(
    pl.kernel,
    out_type=jax.ShapeDtypeStruct((NC, N_PAD, TBW), jnp.float32),
    mesh=_mesh,
    compiler_params=pltpu.CompilerParams(use_tc_tiling_on_sc=False),
    scratch_types=[
        pltpu.VMEM((CPT0, CHUNK), jnp.int32),
        pltpu.VMEM((CPT0, CHUNK), jnp.int32),
        pltpu.VMEM((CHUNK, TBW), jnp.float32),
        pltpu.VMEM((CHUNK, TBW), jnp.float32),
        pltpu.VMEM_SHARED((N_PAD, TBW), jnp.float32),
        pltpu.SemaphoreType.DMA,
        pltpu.SemaphoreType.DMA,
    ],
)
def _edge_agg(table_hbm, edges_hbm, zeros_hbm, out_hbm,
              src_v, dst_v, rows0, rows1, acc, sem0, sem1):
    c = lax.axis_index("c")
    s = lax.axis_index("s")
    row0 = s * ROWS_PT
    pltpu.sync_copy(zeros_hbm.at[pl.ds(row0, ROWS_PT)],
                    acc.at[pl.ds(row0, ROWS_PT)])

    def _run(base, cpt):
        pltpu.sync_copy(edges_hbm.at[0, pl.ds(base, cpt)],
                        src_v.at[pl.ds(0, cpt)])
        pltpu.sync_copy(edges_hbm.at[1, pl.ds(base, cpt)],
                        dst_v.at[pl.ds(0, cpt)])
        plsc.subcore_barrier()

        @pl.loop(0, cpt, step=2)
        def _(j):
            g0 = pltpu.async_copy(table_hbm.at[src_v.at[j]], rows0, sem0)
            g1 = pltpu.async_copy(table_hbm.at[src_v.at[j + 1]], rows1, sem1)
            g0.wait()
            pltpu.sync_copy(rows0, acc.at[dst_v.at[j]], add=True)
            g1.wait()
            pltpu.sync_copy(rows1, acc.at[dst_v.at[j + 1]], add=True)

    @pl.when(c == 0)
    def _():
        _run(s * CPT0, CPT0)

    @pl.when(c != 0)
    def _():
        _run(NS * CPT0 + s * CPT1, CPT1)

    plsc.subcore_barrier()
    pltpu.sync_copy(acc.at[pl.ds(row0, ROWS_PT)],
                    out_hbm.at[c, pl.ds(row0, ROWS_PT)])


BLK = 2000
GRID = N // BLK


def _pre_body(x_ref, wl_ref, wr_ref, b1_ref, t1_ref, r1_ref):
    xb = x_ref[...]
    t = jnp.dot(xb, wl_ref[...], preferred_element_type=jnp.float32)
    col = lax.broadcasted_iota(jnp.int32, (BLK, TBW), 1)
    t1_ref[...] = jnp.where(col == HID, t + 1.0, t)
    r1_ref[...] = (jnp.dot(xb, wr_ref[...], preferred_element_type=jnp.float32)
                   + b1_ref[...])


_pre = pl.pallas_call(
    _pre_body,
    grid=(GRID,),
    in_specs=[pl.BlockSpec((BLK, D), lambda i: (i, 0)),
              pl.BlockSpec((D, TBW), lambda i: (0, 0)),
              pl.BlockSpec((D, HID), lambda i: (0, 0)),
              pl.BlockSpec((1, HID), lambda i: (0, 0))],
    out_specs=[pl.BlockSpec((BLK, TBW), lambda i: (i, 0)),
               pl.BlockSpec((BLK, HID), lambda i: (i, 0))],
    out_shape=[jax.ShapeDtypeStruct((N_PAD, TBW), jnp.float32),
               jax.ShapeDtypeStruct((N_PAD, HID), jnp.float32)],
)


def _mid_body(p_ref, r1_ref, wl_ref, wr_ref, b2_ref, t2_ref, r2_ref, den_ref):
    ssum = p_ref[0] + p_ref[1]
    den = jnp.maximum(ssum[:, HID:HID + 1], 1.0)
    h = ssum[:, :HID] / den + r1_ref[...]
    h = jnp.where(h > 0, h, jnp.exp(jnp.minimum(h, 0.0)) - 1.0)
    t2_ref[...] = jnp.dot(h, wl_ref[...], preferred_element_type=jnp.float32)
    r2_ref[...] = (jnp.dot(h, wr_ref[...], preferred_element_type=jnp.float32)
                   + b2_ref[...])
    den_ref[...] = den


_mid = pl.pallas_call(
    _mid_body,
    grid=(GRID,),
    in_specs=[pl.BlockSpec((NC, BLK, TBW), lambda i: (0, i, 0)),
              pl.BlockSpec((BLK, HID), lambda i: (i, 0)),
              pl.BlockSpec((HID, NCLS), lambda i: (0, 0)),
              pl.BlockSpec((HID, NCLS), lambda i: (0, 0)),
              pl.BlockSpec((1, NCLS), lambda i: (0, 0))],
    out_specs=[pl.BlockSpec((BLK, NCLS), lambda i: (i, 0)),
               pl.BlockSpec((BLK, NCLS), lambda i: (i, 0)),
               pl.BlockSpec((BLK, 1), lambda i: (i, 0))],
    out_shape=[jax.ShapeDtypeStruct((N_PAD, NCLS), jnp.float32),
               jax.ShapeDtypeStruct((N_PAD, NCLS), jnp.float32),
               jax.ShapeDtypeStruct((N_PAD, 1), jnp.float32)],
)


def _fin_body(p_ref, den_ref, r2_ref, o_ref):
    z = (p_ref[0] + p_ref[1]) / den_ref[...] + r2_ref[...]
    m = jnp.max(z, axis=1, keepdims=True)
    e = jnp.exp(z - m)
    lse = jnp.log(jnp.sum(e, axis=1, keepdims=True))
    o_ref[...] = z - m - lse


_fin = pl.pallas_call(
    _fin_body,
    grid=(GRID,),
    in_specs=[pl.BlockSpec((NC, BLK, NCLS), lambda i: (0, i, 0)),
              pl.BlockSpec((BLK, 1), lambda i: (i, 0)),
              pl.BlockSpec((BLK, NCLS), lambda i: (i, 0))],
    out_specs=pl.BlockSpec((BLK, NCLS), lambda i: (i, 0)),
    out_shape=jax.ShapeDtypeStruct((N, NCLS), jnp.float32),
)


def kernel(x, edge_index, W1l, W1r, b1, W2l, W2r, b2):
    ei = edge_index.astype(jnp.int32)
    ei4 = jnp.pad(ei, ((0, 0), (0, E_PAD - E)),
                  constant_values=N).reshape(2, NCHUNKS, CHUNK)
    wl1 = jnp.pad(W1l, ((0, 0), (0, TBW - HID)))
    zeros_tab = jnp.zeros((N_PAD, TBW), jnp.float32)

    T1, R1 = _pre(x, wl1, W1r, b1.reshape(1, HID))
    P1 = _edge_agg(T1, ei4, zeros_tab)
    T2, R2, den = _mid(P1, R1, W2l, W2r, b2.reshape(1, NCLS))
    P2 = _edge_agg(T2, ei4, zeros_tab)
    return _fin(P2, den, R2)


# speedup vs baseline: 19.8821x; 1.1811x over previous
import functools

import jax
import jax.numpy as jnp
from jax import lax
from jax.experimental import pallas as pl
from jax.experimental.pallas import tpu as pltpu
from jax.experimental.pallas import tpu_sc as plsc

N = 10000
D = 128
HID = 8
NCLS = 16
TBW = 16
N_PAD = 10240
E = 320000
NC = 2
NS = 16
NW = NC * NS
CHUNK = 128
CPT0 = 96
CPT1 = 64
NCHUNKS = NS * (CPT0 + CPT1)
E_PAD = NCHUNKS * CHUNK
ROWS_PT = N_PAD // NS

_mesh = plsc.VectorSubcoreMesh(core_axis_name="c", subcore_axis_name="s")


@functools.partial(
    pl.kernel,
    out_type=jax.ShapeDtypeStruct((NC, N_PAD, TBW), jnp.float32),
    mesh=_mesh,
    compiler_params=pltpu.CompilerParams(use_tc_tiling_on_sc=False),
    scratch_types=[
        pltpu.VMEM((CPT0, CHUNK), jnp.int32),
        pltpu.VMEM((CPT0, CHUNK), jnp.int32),
        [pltpu.VMEM((CHUNK, TBW), jnp.float32)] * 8,
        pltpu.VMEM_SHARED((N_PAD, TBW), jnp.float32),
        [pltpu.SemaphoreType.DMA] * 4,
    ],
)
def _edge_agg(table_hbm, edges_hbm, zeros_hbm, out_hbm,
              src_v, dst_v, rows, acc, sems):
    c = lax.axis_index("c")
    s = lax.axis_index("s")
    row0 = s * ROWS_PT
    pltpu.sync_copy(zeros_hbm.at[pl.ds(row0, ROWS_PT)],
                    acc.at[pl.ds(row0, ROWS_PT)])

    K = 4
    bufs = (rows[:K], rows[K:])
    gsem = (sems[0], sems[1])
    ssem = (sems[2], sems[3])

    def _fire_g(g, j):
        for k in range(K):
            pltpu.async_copy(table_hbm.at[src_v.at[j + k]], bufs[g][k],
                             gsem[g])

    def _drain_g(g):
        for k in range(K):
            pltpu.make_async_copy(table_hbm.at[src_v.at[0]], bufs[g][k],
                                  gsem[g]).wait()

    def _fire_s(g, j):
        for k in range(K):
            pltpu.async_copy(bufs[g][k], acc.at[dst_v.at[j + k]], ssem[g],
                             add=True)

    def _drain_s(g):
        for k in range(K):
            pltpu.make_async_copy(bufs[g][k], acc.at[dst_v.at[0]],
                                  ssem[g]).wait()

    def _run(base, cpt):
        pltpu.sync_copy(edges_hbm.at[0, pl.ds(base, cpt)],
                        src_v.at[pl.ds(0, cpt)])
        pltpu.sync_copy(edges_hbm.at[1, pl.ds(base, cpt)],
                        dst_v.at[pl.ds(0, cpt)])
        plsc.subcore_barrier()

        _fire_g(0, 0)

        @pl.loop(0, cpt - 2 * K, step=2 * K)
        def _(j):
            _fire_g(1, j + K)
            _drain_g(0)
            _fire_s(0, j)
            _drain_s(0)
            _fire_g(0, j + 2 * K)
            _drain_g(1)
            _fire_s(1, j + K)
            _drain_s(1)

        _fire_g(1, cpt - K)
        _drain_g(0)
        _fire_s(0, cpt - 2 * K)
        _drain_s(0)
        _drain_g(1)
        _fire_s(1, cpt - K)
        _drain_s(1)

    @pl.when(c == 0)
    def _():
        _run(s * CPT0, CPT0)

    @pl.when(c != 0)
    def _():
        _run(NS * CPT0 + s * CPT1, CPT1)

    plsc.subcore_barrier()
    pltpu.sync_copy(acc.at[pl.ds(row0, ROWS_PT)],
                    out_hbm.at[c, pl.ds(row0, ROWS_PT)])


BLK = 2000
GRID = N // BLK


def _pre_body(x_ref, wl_ref, wr_ref, b1_ref, t1_ref, r1_ref):
    xb = x_ref[...]
    t = jnp.dot(xb, wl_ref[...], preferred_element_type=jnp.float32)
    col = lax.broadcasted_iota(jnp.int32, (BLK, TBW), 1)
    t1_ref[...] = jnp.where(col == HID, t + 1.0, t)
    r1_ref[...] = (jnp.dot(xb, wr_ref[...], preferred_element_type=jnp.float32)
                   + b1_ref[...])


_pre = pl.pallas_call(
    _pre_body,
    grid=(GRID,),
    in_specs=[pl.BlockSpec((BLK, D), lambda i: (i, 0)),
              pl.BlockSpec((D, TBW), lambda i: (0, 0)),
              pl.BlockSpec((D, HID), lambda i: (0, 0)),
              pl.BlockSpec((1, HID), lambda i: (0, 0))],
    out_specs=[pl.BlockSpec((BLK, TBW), lambda i: (i, 0)),
               pl.BlockSpec((BLK, HID), lambda i: (i, 0))],
    out_shape=[jax.ShapeDtypeStruct((N_PAD, TBW), jnp.float32),
               jax.ShapeDtypeStruct((N_PAD, HID), jnp.float32)],
)


def _mid_body(p_ref, r1_ref, wl_ref, wr_ref, b2_ref, t2_ref, r2_ref, den_ref):
    ssum = p_ref[0] + p_ref[1]
    den = jnp.maximum(ssum[:, HID:HID + 1], 1.0)
    h = ssum[:, :HID] / den + r1_ref[...]
    h = jnp.where(h > 0, h, jnp.exp(jnp.minimum(h, 0.0)) - 1.0)
    t2_ref[...] = jnp.dot(h, wl_ref[...], preferred_element_type=jnp.float32)
    r2_ref[...] = (jnp.dot(h, wr_ref[...], preferred_element_type=jnp.float32)
                   + b2_ref[...])
    den_ref[...] = den


_mid = pl.pallas_call(
    _mid_body,
    grid=(GRID,),
    in_specs=[pl.BlockSpec((NC, BLK, TBW), lambda i: (0, i, 0)),
              pl.BlockSpec((BLK, HID), lambda i: (i, 0)),
              pl.BlockSpec((HID, NCLS), lambda i: (0, 0)),
              pl.BlockSpec((HID, NCLS), lambda i: (0, 0)),
              pl.BlockSpec((1, NCLS), lambda i: (0, 0))],
    out_specs=[pl.BlockSpec((BLK, NCLS), lambda i: (i, 0)),
               pl.BlockSpec((BLK, NCLS), lambda i: (i, 0)),
               pl.BlockSpec((BLK, 1), lambda i: (i, 0))],
    out_shape=[jax.ShapeDtypeStruct((N_PAD, NCLS), jnp.float32),
               jax.ShapeDtypeStruct((N_PAD, NCLS), jnp.float32),
               jax.ShapeDtypeStruct((N_PAD, 1), jnp.float32)],
)


def _fin_body(p_ref, den_ref, r2_ref, o_ref):
    z = (p_ref[0] + p_ref[1]) / den_ref[...] + r2_ref[...]
    m = jnp.max(z, axis=1, keepdims=True)
    e = jnp.exp(z - m)
    lse = jnp.log(jnp.sum(e, axis=1, keepdims=True))
    o_ref[...] = z - m - lse


_fin = pl.pallas_call(
    _fin_body,
    grid=(GRID,),
    in_specs=[pl.BlockSpec((NC, BLK, NCLS), lambda i: (0, i, 0)),
              pl.BlockSpec((BLK, 1), lambda i: (i, 0)),
              pl.BlockSpec((BLK, NCLS), lambda i: (i, 0))],
    out_specs=pl.BlockSpec((BLK, NCLS), lambda i: (i, 0)),
    out_shape=jax.ShapeDtypeStruct((N, NCLS), jnp.float32),
)


def kernel(x, edge_index, W1l, W1r, b1, W2l, W2r, b2):
    ei = edge_index.astype(jnp.int32)
    ei4 = jnp.pad(ei, ((0, 0), (0, E_PAD - E)),
                  constant_values=N).reshape(2, NCHUNKS, CHUNK)
    wl1 = jnp.pad(W1l, ((0, 0), (0, TBW - HID)))
    zeros_tab = jnp.zeros((N_PAD, TBW), jnp.float32)

    T1, R1 = _pre(x, wl1, W1r, b1.reshape(1, HID))
    P1 = _edge_agg(T1, ei4, zeros_tab)
    T2, R2, den = _mid(P1, R1, W2l, W2r, b2.reshape(1, NCLS))
    P2 = _edge_agg(T2, ei4, zeros_tab)
    return _fin(P2, den, R2)
